# baseline (speedup 1.0000x reference)
import jax
import jax.numpy as jnp
from jax import lax
from jax.experimental import pallas as pl
from jax.experimental.pallas import tpu as pltpu
from jax.experimental.pallas import tpu_sc as plsc

_B = 16384
_D = 128
_MARGIN = 1.0
_NC = 2
_NS = 16
_NW = _NC * _NS
_SLICES = (8192, 8192)
_C = 32


def _gather6(gbase, bs, h_ids, r_ids, t_ids, ent_emb, rel_emb, ent_tr, rel_tr):
    bpw = bs // _NW
    nch = bpw // _C
    npair = nch // 2
    mesh = plsc.VectorSubcoreMesh(
        core_axis_name="c", subcore_axis_name="s",
        num_cores=_NC, num_subcores=_NS)
    row = jax.ShapeDtypeStruct((bs, _D), jnp.float32)

    def body(h_ref, r_ref, t_ref, ee_ref, re_ref, et_ref, rt_ref,
             oh, orl, ot, ohtr, ortr, ottr,
             hidx, ridx, tidx, bufs, sh, gs0, gs1, xs0, xs1, hs0, hs1):
        sid = lax.axis_index("s")
        wid = sid * _NC + lax.axis_index("c")
        base = wid * bpw
        pltpu.sync_copy(h_ref.at[pl.ds(gbase + base, bpw)], hidx)
        pltpu.sync_copy(r_ref.at[pl.ds(gbase + base, bpw)], ridx)
        pltpu.sync_copy(t_ref.at[pl.ds(gbase + base, bpw)], tidx)
        jobs = ((ee_ref, hidx, oh), (re_ref, ridx, orl), (ee_ref, tidx, ot),
                (et_ref, hidx, ohtr), (rt_ref, ridx, ortr), (et_ref, tidx, ottr))

        def g_desc(c, p, sem):
            off = c * _C
            return [pltpu.make_async_copy(tbl.at[idx.at[pl.ds(off, _C)]],
                                          bufs.at[p, j], sem)
                    for j, (tbl, idx, _) in enumerate(jobs)]

        def x_desc(p, sem):
            return [pltpu.make_async_copy(bufs.at[p, j], sh.at[p, j, sid], sem)
                    for j in range(6)]

        def h_desc(c, p, sem):
            off = c * _C
            return [pltpu.make_async_copy(sh.at[p, j, sid],
                                          out.at[pl.ds(base + off, _C)], sem)
                    for j, (_, _, out) in enumerate(jobs)]

        for dsc in g_desc(0, 0, gs0):
            dsc.start()

        def pair(k, carry):
            c0 = 2 * k
            c1 = c0 + 1
            for dsc in g_desc(c0, 0, gs0):
                dsc.wait()

            @pl.when(k > 0)
            def _():
                for dsc in x_desc(1, xs1):
                    dsc.wait()
                for dsc in h_desc(c1 - 2, 1, hs1):
                    dsc.start()

            for dsc in g_desc(c1, 1, gs1):
                dsc.start()

            @pl.when(k > 0)
            def _():
                for dsc in h_desc(c0 - 2, 0, hs0):
                    dsc.wait()

            for dsc in x_desc(0, xs0):
                dsc.start()
            for dsc in g_desc(c1, 1, gs1):
                dsc.wait()

            @pl.when(k + 1 < npair)
            def _():
                for dsc in x_desc(0, xs0):
                    dsc.wait()
                for dsc in h_desc(c0, 0, hs0):
                    dsc.start()
                for dsc in g_desc(c0 + 2, 0, gs0):
                    dsc.start()

            @pl.when(k > 0)
            def _():
                for dsc in h_desc(c1 - 2, 1, hs1):
                    dsc.wait()

            for dsc in x_desc(1, xs1):
                dsc.start()
            return carry

        lax.fori_loop(0, npair, pair, 0)
        for dsc in x_desc(0, xs0):
            dsc.wait()
        for dsc in h_desc(nch - 2, 0, hs0):
            dsc.start()
        for dsc in x_desc(1, xs1):
            dsc.wait()
        for dsc in h_desc(nch - 1, 1, hs1):
            dsc.start()
        for dsc in h_desc(nch - 2, 0, hs0):
            dsc.wait()
        for dsc in h_desc(nch - 1, 1, hs1):
            dsc.wait()

    fn = pl.kernel(
        body,
        out_type=(row,) * 6,
        mesh=mesh,
        scratch_types=[
            pltpu.VMEM((bpw,), jnp.int32),
            pltpu.VMEM((bpw,), jnp.int32),
            pltpu.VMEM((bpw,), jnp.int32),
            pltpu.VMEM((2, 6, _C, _D), jnp.float32),
            pltpu.VMEM_SHARED((2, 6, _NS, _C, _D), jnp.float32),
            pltpu.SemaphoreType.DMA,
            pltpu.SemaphoreType.DMA,
            pltpu.SemaphoreType.DMA,
            pltpu.SemaphoreType.DMA,
            pltpu.SemaphoreType.DMA,
            pltpu.SemaphoreType.DMA,
        ],
    )
    return fn(h_ids, r_ids, t_ids, ent_emb, rel_emb, ent_tr, rel_tr)


def _l2n(x):
    n = jnp.sqrt(jnp.sum(x * x, axis=-1, keepdims=True))
    return x / jnp.maximum(n, 1e-12)


def _score_body(h_ref, r_ref, t_ref, htr_ref, rtr_ref, ttr_ref, o_ref):
    head = h_ref[...]
    rel = r_ref[...]
    tail = t_ref[...]
    h_tr = htr_ref[...]
    r_tr = rtr_ref[...]
    t_tr = ttr_ref[...]
    hh = _l2n(head + jnp.sum(head * h_tr, axis=-1, keepdims=True) * r_tr)
    tt = _l2n(tail + jnp.sum(tail * t_tr, axis=-1, keepdims=True) * r_tr)
    rr = _l2n(rel)
    o_ref[...] = _MARGIN - jnp.sum(jnp.abs(hh + rr - tt), axis=-1)


_ROWS_PER_BLOCK = 1024


def _score(bs, gh, gr, gt, ghtr, grtr, gttr):
    in_spec = pl.BlockSpec((_ROWS_PER_BLOCK, _D), lambda i: (i, 0))
    return pl.pallas_call(
        _score_body,
        grid=(bs // _ROWS_PER_BLOCK,),
        in_specs=[in_spec] * 6,
        out_specs=pl.BlockSpec((_ROWS_PER_BLOCK,), lambda i: (i,)),
        out_shape=jax.ShapeDtypeStruct((bs,), jnp.float32),
    )(gh, gr, gt, ghtr, grtr, gttr)


def kernel(sample, ent_embeddings, rel_embeddings, ent_transfer, rel_transfer):
    h_ids = sample[:, 0]
    r_ids = sample[:, 1]
    t_ids = sample[:, 2]
    scores = []
    gbase = 0
    for bs in _SLICES:
        g = _gather6(gbase, bs, h_ids, r_ids, t_ids,
                     ent_embeddings, rel_embeddings, ent_transfer, rel_transfer)
        scores.append(_score(bs, *g))
        gbase += bs
    return jnp.concatenate(scores)


# speedup vs baseline: 1.0066x; 1.0066x over previous
import jax
import jax.numpy as jnp
from jax import lax
from jax.experimental import pallas as pl
from jax.experimental.pallas import tpu as pltpu
from jax.experimental.pallas import tpu_sc as plsc

_B = 16384
_D = 128
_MARGIN = 1.0
_NC = 2
_NS = 16
_NW = _NC * _NS
_SLICES = (8192, 8192)
_C = 64


def _gather6(gbase, bs, h_ids, r_ids, t_ids, ent_emb, rel_emb, ent_tr, rel_tr):
    bpw = bs // _NW
    nch = bpw // _C
    npair = nch // 2
    mesh = plsc.VectorSubcoreMesh(
        core_axis_name="c", subcore_axis_name="s",
        num_cores=_NC, num_subcores=_NS)
    row = jax.ShapeDtypeStruct((bs, _D), jnp.float32)

    def body(h_ref, r_ref, t_ref, ee_ref, re_ref, et_ref, rt_ref,
             oh, orl, ot, ohtr, ortr, ottr,
             hidx, ridx, tidx, bufs, sh, gs0, gs1, ws0, ws1, xs, hs):
        sid = lax.axis_index("s")
        wid = sid * _NC + lax.axis_index("c")
        base = wid * bpw
        pltpu.sync_copy(h_ref.at[pl.ds(gbase + base, bpw)], hidx)
        pltpu.sync_copy(r_ref.at[pl.ds(gbase + base, bpw)], ridx)
        pltpu.sync_copy(t_ref.at[pl.ds(gbase + base, bpw)], tidx)
        jobs = ((ee_ref, hidx, oh), (re_ref, ridx, orl), (ee_ref, tidx, ot),
                (et_ref, hidx, ohtr), (rt_ref, ridx, ortr), (et_ref, tidx, ottr))

        def g_desc(c, p, sem):
            off = c * _C
            return [pltpu.make_async_copy(tbl.at[idx.at[pl.ds(off, _C)]],
                                          bufs.at[p, j], sem)
                    for j, (tbl, idx, _) in enumerate(jobs)]

        def wb_desc(c, p, sem):
            off = c * _C
            return [pltpu.make_async_copy(bufs.at[p, j],
                                          jobs[j][2].at[pl.ds(base + off, _C)],
                                          sem)
                    for j in range(3)]

        def x_desc(p, sem):
            return [pltpu.make_async_copy(bufs.at[p, j], sh.at[j - 3, sid], sem)
                    for j in range(3, 6)]

        def h_desc(c, sem):
            off = c * _C
            return [pltpu.make_async_copy(sh.at[j - 3, sid],
                                          jobs[j][2].at[pl.ds(base + off, _C)],
                                          sem)
                    for j in range(3, 6)]

        for dsc in g_desc(0, 0, gs0):
            dsc.start()

        def pair(k, carry):
            c0 = 2 * k
            c1 = c0 + 1
            for dsc in g_desc(c0, 0, gs0):
                dsc.wait()

            @pl.when(k > 0)
            def _():
                for dsc in wb_desc(c1 - 2, 1, ws1):
                    dsc.wait()
                for dsc in x_desc(1, xs):
                    dsc.wait()
                for dsc in h_desc(c1 - 2, hs):
                    dsc.start()

            for dsc in g_desc(c1, 1, gs1):
                dsc.start()

            @pl.when(k > 0)
            def _():
                for dsc in h_desc(c1 - 2, hs):
                    dsc.wait()

            for dsc in x_desc(0, xs):
                dsc.start()
            for dsc in wb_desc(c0, 0, ws0):
                dsc.start()
            for dsc in g_desc(c1, 1, gs1):
                dsc.wait()

            @pl.when(k + 1 < npair)
            def _():
                for dsc in wb_desc(c0, 0, ws0):
                    dsc.wait()
                for dsc in x_desc(0, xs):
                    dsc.wait()
                for dsc in h_desc(c0, hs):
                    dsc.start()
                for dsc in g_desc(c0 + 2, 0, gs0):
                    dsc.start()
                for dsc in h_desc(c0, hs):
                    dsc.wait()
                for dsc in x_desc(1, xs):
                    dsc.start()

            for dsc in wb_desc(c1, 1, ws1):
                dsc.start()
            return carry

        lax.fori_loop(0, npair, pair, 0)
        for dsc in wb_desc(nch - 2, 0, ws0):
            dsc.wait()
        for dsc in x_desc(0, xs):
            dsc.wait()
        for dsc in h_desc(nch - 2, hs):
            dsc.start()
        for dsc in h_desc(nch - 2, hs):
            dsc.wait()
        for dsc in x_desc(1, xs):
            dsc.start()
        for dsc in x_desc(1, xs):
            dsc.wait()
        for dsc in h_desc(nch - 1, hs):
            dsc.start()
        for dsc in wb_desc(nch - 1, 1, ws1):
            dsc.wait()
        for dsc in h_desc(nch - 1, hs):
            dsc.wait()

    fn = pl.kernel(
        body,
        out_type=(row,) * 6,
        mesh=mesh,
        scratch_types=[
            pltpu.VMEM((bpw,), jnp.int32),
            pltpu.VMEM((bpw,), jnp.int32),
            pltpu.VMEM((bpw,), jnp.int32),
            pltpu.VMEM((2, 6, _C, _D), jnp.float32),
            pltpu.VMEM_SHARED((3, _NS, _C, _D), jnp.float32),
            pltpu.SemaphoreType.DMA,
            pltpu.SemaphoreType.DMA,
            pltpu.SemaphoreType.DMA,
            pltpu.SemaphoreType.DMA,
            pltpu.SemaphoreType.DMA,
            pltpu.SemaphoreType.DMA,
        ],
    )
    return fn(h_ids, r_ids, t_ids, ent_emb, rel_emb, ent_tr, rel_tr)


def _l2n(x):
    n = jnp.sqrt(jnp.sum(x * x, axis=-1, keepdims=True))
    return x / jnp.maximum(n, 1e-12)


def _score_body(h_ref, r_ref, t_ref, htr_ref, rtr_ref, ttr_ref, o_ref):
    head = h_ref[...]
    rel = r_ref[...]
    tail = t_ref[...]
    h_tr = htr_ref[...]
    r_tr = rtr_ref[...]
    t_tr = ttr_ref[...]
    hh = _l2n(head + jnp.sum(head * h_tr, axis=-1, keepdims=True) * r_tr)
    tt = _l2n(tail + jnp.sum(tail * t_tr, axis=-1, keepdims=True) * r_tr)
    rr = _l2n(rel)
    o_ref[...] = _MARGIN - jnp.sum(jnp.abs(hh + rr - tt), axis=-1)


_ROWS_PER_BLOCK = 1024


def _score(bs, gh, gr, gt, ghtr, grtr, gttr):
    in_spec = pl.BlockSpec((_ROWS_PER_BLOCK, _D), lambda i: (i, 0))
    return pl.pallas_call(
        _score_body,
        grid=(bs // _ROWS_PER_BLOCK,),
        in_specs=[in_spec] * 6,
        out_specs=pl.BlockSpec((_ROWS_PER_BLOCK,), lambda i: (i,)),
        out_shape=jax.ShapeDtypeStruct((bs,), jnp.float32),
    )(gh, gr, gt, ghtr, grtr, gttr)


def kernel(sample, ent_embeddings, rel_embeddings, ent_transfer, rel_transfer):
    h_ids = sample[:, 0]
    r_ids = sample[:, 1]
    t_ids = sample[:, 2]
    scores = []
    gbase = 0
    for bs in _SLICES:
        g = _gather6(gbase, bs, h_ids, r_ids, t_ids,
                     ent_embeddings, rel_embeddings, ent_transfer, rel_transfer)
        scores.append(_score(bs, *g))
        gbase += bs
    return jnp.concatenate(scores)


# speedup vs baseline: 1.0219x; 1.0152x over previous
import jax
import jax.numpy as jnp
from jax import lax
from jax.experimental import pallas as pl
from jax.experimental.pallas import tpu as pltpu
from jax.experimental.pallas import tpu_sc as plsc

_B = 16384
_D = 128
_MARGIN = 1.0
_NC = 2
_NS = 16
_NW = _NC * _NS
_SLICES = (8192, 8192)
_C = 64


def _gather6(gbase, bs, h_ids, r_ids, t_ids, ent_emb, rel_emb, ent_tr, rel_tr):
    bpw = bs // _NW
    nch = bpw // _C
    npair = nch // 2
    mesh = plsc.VectorSubcoreMesh(
        core_axis_name="c", subcore_axis_name="s",
        num_cores=_NC, num_subcores=_NS)
    row = jax.ShapeDtypeStruct((bs, _D), jnp.float32)

    def body(h_ref, r_ref, t_ref, ee_ref, re_ref, et_ref, rt_ref,
             oh, orl, ot, ohtr, ortr, ottr,
             hidx, ridx, tidx, bufs, gs0, gs1, ws0, ws1):
        wid = lax.axis_index("s") * _NC + lax.axis_index("c")
        base = wid * bpw
        pltpu.sync_copy(h_ref.at[pl.ds(gbase + base, bpw)], hidx)
        pltpu.sync_copy(r_ref.at[pl.ds(gbase + base, bpw)], ridx)
        pltpu.sync_copy(t_ref.at[pl.ds(gbase + base, bpw)], tidx)
        jobs = ((ee_ref, hidx, oh), (re_ref, ridx, orl), (ee_ref, tidx, ot),
                (et_ref, hidx, ohtr), (rt_ref, ridx, ortr), (et_ref, tidx, ottr))

        def g_desc(c, p, sem):
            off = c * _C
            return [pltpu.make_async_copy(tbl.at[idx.at[pl.ds(off, _C)]],
                                          bufs.at[p, j], sem)
                    for j, (tbl, idx, _) in enumerate(jobs)]

        def wb_desc(c, p, sem):
            off = c * _C
            return [pltpu.make_async_copy(bufs.at[p, j],
                                          out.at[pl.ds(base + off, _C)], sem)
                    for j, (_, _, out) in enumerate(jobs)]

        for dsc in g_desc(0, 0, gs0):
            dsc.start()

        def pair(k, carry):
            c0 = 2 * k
            c1 = c0 + 1
            for dsc in g_desc(c0, 0, gs0):
                dsc.wait()

            @pl.when(k > 0)
            def _():
                for dsc in wb_desc(c1 - 2, 1, ws1):
                    dsc.wait()

            for dsc in g_desc(c1, 1, gs1):
                dsc.start()
            for dsc in wb_desc(c0, 0, ws0):
                dsc.start()
            for dsc in g_desc(c1, 1, gs1):
                dsc.wait()

            @pl.when(k + 1 < npair)
            def _():
                for dsc in wb_desc(c0, 0, ws0):
                    dsc.wait()
                for dsc in g_desc(c0 + 2, 0, gs0):
                    dsc.start()

            for dsc in wb_desc(c1, 1, ws1):
                dsc.start()
            return carry

        lax.fori_loop(0, npair, pair, 0)
        for dsc in wb_desc(nch - 2, 0, ws0):
            dsc.wait()
        for dsc in wb_desc(nch - 1, 1, ws1):
            dsc.wait()

    fn = pl.kernel(
        body,
        out_type=(row,) * 6,
        mesh=mesh,
        scratch_types=[
            pltpu.VMEM((bpw,), jnp.int32),
            pltpu.VMEM((bpw,), jnp.int32),
            pltpu.VMEM((bpw,), jnp.int32),
            pltpu.VMEM((2, 6, _C, _D), jnp.float32),
            pltpu.SemaphoreType.DMA,
            pltpu.SemaphoreType.DMA,
            pltpu.SemaphoreType.DMA,
            pltpu.SemaphoreType.DMA,
        ],
    )
    return fn(h_ids, r_ids, t_ids, ent_emb, rel_emb, ent_tr, rel_tr)


def _l2n(x):
    n = jnp.sqrt(jnp.sum(x * x, axis=-1, keepdims=True))
    return x / jnp.maximum(n, 1e-12)


def _score_body(h_ref, r_ref, t_ref, htr_ref, rtr_ref, ttr_ref, o_ref):
    head = h_ref[...]
    rel = r_ref[...]
    tail = t_ref[...]
    h_tr = htr_ref[...]
    r_tr = rtr_ref[...]
    t_tr = ttr_ref[...]
    hh = _l2n(head + jnp.sum(head * h_tr, axis=-1, keepdims=True) * r_tr)
    tt = _l2n(tail + jnp.sum(tail * t_tr, axis=-1, keepdims=True) * r_tr)
    rr = _l2n(rel)
    o_ref[...] = _MARGIN - jnp.sum(jnp.abs(hh + rr - tt), axis=-1)


_ROWS_PER_BLOCK = 1024


def _score(bs, gh, gr, gt, ghtr, grtr, gttr):
    in_spec = pl.BlockSpec((_ROWS_PER_BLOCK, _D), lambda i: (i, 0))
    return pl.pallas_call(
        _score_body,
        grid=(bs // _ROWS_PER_BLOCK,),
        in_specs=[in_spec] * 6,
        out_specs=pl.BlockSpec((_ROWS_PER_BLOCK,), lambda i: (i,)),
        out_shape=jax.ShapeDtypeStruct((bs,), jnp.float32),
    )(gh, gr, gt, ghtr, grtr, gttr)


def kernel(sample, ent_embeddings, rel_embeddings, ent_transfer, rel_transfer):
    h_ids = sample[:, 0]
    r_ids = sample[:, 1]
    t_ids = sample[:, 2]
    scores = []
    gbase = 0
    for bs in _SLICES:
        g = _gather6(gbase, bs, h_ids, r_ids, t_ids,
                     ent_embeddings, rel_embeddings, ent_transfer, rel_transfer)
        scores.append(_score(bs, *g))
        gbase += bs
    return jnp.concatenate(scores)
